# Initial kernel scaffold; baseline (speedup 1.0000x reference)
import functools

import jax
import jax.numpy as jnp
from jax import lax
from jax.experimental import pallas as pl
from jax.experimental.pallas import tpu as pltpu
from jax.experimental.pallas import tpu_sc as plsc

EPSV = 1e-4
NGRAPH = 64
NWORK = 32
CHUNK = 128
GROUPS = CHUNK // 16
NEDGE_SIDE = 320000
NEDGE = 2 * NEDGE_SIDE
NCHUNK = NEDGE // CHUNK
POS_CHUNKS = NEDGE_SIDE // CHUNK
NNODE = 10000
DIM = 128
LN2 = 0.6931471805599453


def _neglog_eps_sigmoid(w):
    wc = jnp.clip(w, -80.0, 80.0)
    u = jnp.exp(-wc)
    t = EPSV + 1.0 / (1.0 + u)
    bits = plsc.bitcast(t, jnp.int32)
    e = (bits >> 23) & 0xFF
    mbits = (bits & 0x7FFFFF) | 0x3F800000
    m = plsc.bitcast(mbits, jnp.float32)
    big = m > 1.4142135623730951
    m2 = jnp.where(big, m * 0.5, m)
    ef = (e - 127).astype(jnp.float32) + jnp.where(big, 1.0, 0.0)
    s = (m2 - 1.0) / (m2 + 1.0)
    s2 = s * s
    lnm = 2.0 * s * (1.0 + s2 * (1.0 / 3.0 + s2 * (0.2 + s2 * (1.0 / 7.0))))
    return -(ef * LN2 + lnm)


def _sc_body(x_hbm, src_hbm, dst_hbm, batch_hbm, out_hbm,
             batch_v, srcs_v, dsts_v, srow_v, drow_v, acc_v, sem):
    cid = lax.axis_index("c")
    sid = lax.axis_index("s")
    wid = sid * 2 + cid

    iot = lax.iota(jnp.int32, 16)
    ones = jnp.ones((16,), jnp.float32)
    zeros = jnp.zeros((16,), jnp.float32)

    pltpu.sync_copy(batch_hbm, batch_v)
    for r in range(256):
        acc_v[r, :] = zeros

    nchunks = 156 + (wid < 8).astype(jnp.int32)

    def chunk_body(ci, carry):
        c = wid + ci * NWORK
        eoff = c * CHUNK
        pltpu.sync_copy(src_hbm.at[pl.ds(eoff, CHUNK)], srcs_v)
        pltpu.sync_copy(dst_hbm.at[pl.ds(eoff, CHUNK)], dsts_v)
        h1 = pltpu.async_copy(x_hbm.at[srcs_v], srow_v, sem)
        h2 = pltpu.async_copy(x_hbm.at[dsts_v], drow_v, sem)
        h1.wait()
        h2.wait()

        is_pos = c < POS_CHUNKS
        sgn = jnp.where(is_pos, 1.0, -1.0)
        base_row = jnp.where(is_pos, 0, 128)

        def group_body(g, gcarry):
            e16 = g * 16 + iot
            src16 = plsc.load_gather(srcs_v, [e16])
            seg = plsc.load_gather(batch_v, [src16])
            dot = jnp.zeros((16,), jnp.float32)
            for d in range(DIM):
                dsplat = jnp.full((16,), d, jnp.int32)
                sv = plsc.load_gather(srow_v, [e16, dsplat])
                dv = plsc.load_gather(drow_v, [e16, dsplat])
                dot = dot + sv * dv
            val = _neglog_eps_sigmoid(dot * sgn)
            rows = base_row + seg
            plsc.addupdate_scatter(acc_v, [rows, iot], val)
            plsc.addupdate_scatter(acc_v, [rows + 64, iot], ones)
            return gcarry

        lax.fori_loop(0, GROUPS, group_body, 0)
        return carry

    lax.fori_loop(0, nchunks, chunk_body, 0)
    pltpu.sync_copy(acc_v, out_hbm.at[wid])


def _combine_body(p_ref, o_ref):
    tot = p_ref[pl.ds(0, 256), :]
    for w in range(1, NWORK):
        tot = tot + p_ref[pl.ds(w * 256, 256), :]
    pos_sum = jnp.sum(tot[0:64, :], axis=1, keepdims=True)
    pos_cnt = jnp.sum(tot[64:128, :], axis=1, keepdims=True)
    neg_sum = jnp.sum(tot[128:192, :], axis=1, keepdims=True)
    neg_cnt = jnp.sum(tot[192:256, :], axis=1, keepdims=True)
    pos_mean = pos_sum / jnp.maximum(pos_cnt, 1.0)
    neg_mean = neg_sum / jnp.maximum(neg_cnt, 1.0)
    o_ref[0, 0] = jnp.sum(pos_mean) + jnp.sum(neg_mean)


def kernel(x, pos_edge_index, neg_edge_index, batch):
    pos = pos_edge_index.astype(jnp.int32)
    neg = neg_edge_index.astype(jnp.int32)
    src = jnp.concatenate([pos[0], neg[0]])
    dst = jnp.concatenate([pos[1], neg[1]])
    batch32 = batch.astype(jnp.int32)

    mesh = plsc.VectorSubcoreMesh(core_axis_name="c", subcore_axis_name="s")
    sc = pl.kernel(
        _sc_body,
        out_type=jax.ShapeDtypeStruct((NWORK, 256, 16), jnp.float32),
        mesh=mesh,
        scratch_types=[
            pltpu.VMEM((NNODE,), jnp.int32),
            pltpu.VMEM((CHUNK,), jnp.int32),
            pltpu.VMEM((CHUNK,), jnp.int32),
            pltpu.VMEM((CHUNK, DIM), jnp.float32),
            pltpu.VMEM((CHUNK, DIM), jnp.float32),
            pltpu.VMEM((256, 16), jnp.float32),
            pltpu.SemaphoreType.DMA,
        ],
    )
    parts = sc(x, src, dst, batch32)

    lreg = pl.pallas_call(
        _combine_body,
        out_shape=jax.ShapeDtypeStruct((1, 1), jnp.float32),
    )(parts.reshape(NWORK * 256, 16))
    return lreg[0, 0]


# speedup vs baseline: 2.6108x; 2.6108x over previous
import functools

import jax
import jax.numpy as jnp
from jax import lax
from jax.experimental import pallas as pl
from jax.experimental.pallas import tpu as pltpu
from jax.experimental.pallas import tpu_sc as plsc

EPSV = 1e-4
NGRAPH = 64
NWORK = 32
CHUNK = 128
GROUPS = CHUNK // 16
NEDGE_SIDE = 320000
NEDGE = 2 * NEDGE_SIDE
NCHUNK = NEDGE // CHUNK
POS_CHUNKS = NEDGE_SIDE // CHUNK
NNODE = 10000
DIM = 128
LN2 = 0.6931471805599453


def _neglog_eps_sigmoid(w):
    wc = jnp.clip(w, -80.0, 80.0)
    u = jnp.exp(-wc)
    t = EPSV + 1.0 / (1.0 + u)
    bits = plsc.bitcast(t, jnp.int32)
    e = (bits >> 23) & 0xFF
    mbits = (bits & 0x7FFFFF) | 0x3F800000
    m = plsc.bitcast(mbits, jnp.float32)
    big = m > 1.4142135623730951
    m2 = jnp.where(big, m * 0.5, m)
    ef = (e - 127).astype(jnp.float32) + jnp.where(big, 1.0, 0.0)
    s = (m2 - 1.0) / (m2 + 1.0)
    s2 = s * s
    lnm = 2.0 * s * (1.0 + s2 * (1.0 / 3.0 + s2 * (0.2 + s2 * (1.0 / 7.0))))
    return -(ef * LN2 + lnm)


def _sc_body(x_hbm, src_hbm, dst_hbm, batch_hbm, out_hbm,
             batch_v, srcs_v, dsts_v, srow_v, drow_v, acc_v, sem):
    cid = lax.axis_index("c")
    sid = lax.axis_index("s")
    wid = sid * 2 + cid

    iot = lax.iota(jnp.int32, 16)
    ones = jnp.ones((16,), jnp.float32)
    zeros = jnp.zeros((16,), jnp.float32)

    pltpu.sync_copy(batch_hbm, batch_v)
    for r in range(256):
        acc_v[r, :] = zeros

    nchunks = 156 + (wid < 8).astype(jnp.int32)

    def chunk_body(ci, carry):
        c = wid + ci * NWORK
        eoff = c * CHUNK
        pltpu.sync_copy(src_hbm.at[pl.ds(eoff, CHUNK)], srcs_v)
        pltpu.sync_copy(dst_hbm.at[pl.ds(eoff, CHUNK)], dsts_v)
        h1 = pltpu.async_copy(x_hbm.at[srcs_v], srow_v, sem)
        h2 = pltpu.async_copy(x_hbm.at[dsts_v], drow_v, sem)
        h1.wait()
        h2.wait()

        is_pos = c < POS_CHUNKS
        sgn = jnp.where(is_pos, 1.0, -1.0)
        base_row = jnp.where(is_pos, 0, 128)

        def group_body(g, gcarry):
            e16 = g * 16 + iot
            src16 = plsc.load_gather(srcs_v, [e16])
            seg = plsc.load_gather(batch_v, [src16])
            dot = jnp.zeros((16,), jnp.float32)
            for d in range(DIM):
                dsplat = jnp.full((16,), d, jnp.int32)
                sv = plsc.load_gather(srow_v, [e16, dsplat])
                dv = plsc.load_gather(drow_v, [e16, dsplat])
                dot = dot + sv * dv
            val = _neglog_eps_sigmoid(dot * sgn)
            rows = base_row + seg
            plsc.addupdate_scatter(acc_v, [rows, iot], val)
            plsc.addupdate_scatter(acc_v, [rows + 64, iot], ones)
            return gcarry

        lax.fori_loop(0, GROUPS, group_body, 0)
        return carry

    lax.fori_loop(0, nchunks, chunk_body, 0)
    pltpu.sync_copy(acc_v, out_hbm.at[wid])


def _combine_body(p_ref, o_ref):
    tot = p_ref[pl.ds(0, 256), :]
    for w in range(1, NWORK):
        tot = tot + p_ref[pl.ds(w * 256, 256), :]
    pos_sum = jnp.sum(tot[0:64, :], axis=1, keepdims=True)
    pos_cnt = jnp.sum(tot[64:128, :], axis=1, keepdims=True)
    neg_sum = jnp.sum(tot[128:192, :], axis=1, keepdims=True)
    neg_cnt = jnp.sum(tot[192:256, :], axis=1, keepdims=True)
    pos_mean = pos_sum / jnp.maximum(pos_cnt, 1.0)
    neg_mean = neg_sum / jnp.maximum(neg_cnt, 1.0)
    o_ref[...] = (jnp.sum(pos_mean, keepdims=True)
                  + jnp.sum(neg_mean, keepdims=True))


def kernel(x, pos_edge_index, neg_edge_index, batch):
    pos = pos_edge_index.astype(jnp.int32)
    neg = neg_edge_index.astype(jnp.int32)
    src = jnp.concatenate([pos[0], neg[0]])
    dst = jnp.concatenate([pos[1], neg[1]])
    batch32 = batch.astype(jnp.int32)

    mesh = plsc.VectorSubcoreMesh(core_axis_name="c", subcore_axis_name="s")
    sc = pl.kernel(
        _sc_body,
        out_type=jax.ShapeDtypeStruct((NWORK, 256, 16), jnp.float32),
        mesh=mesh,
        compiler_params=pltpu.CompilerParams(needs_layout_passes=False),
        scratch_types=[
            pltpu.VMEM((NNODE,), jnp.int32),
            pltpu.VMEM((CHUNK,), jnp.int32),
            pltpu.VMEM((CHUNK,), jnp.int32),
            pltpu.VMEM((CHUNK, DIM), jnp.float32),
            pltpu.VMEM((CHUNK, DIM), jnp.float32),
            pltpu.VMEM((256, 16), jnp.float32),
            pltpu.SemaphoreType.DMA,
        ],
    )
    parts = sc(x, src, dst, batch32)

    lreg = pl.pallas_call(
        _combine_body,
        out_shape=jax.ShapeDtypeStruct((1, 1), jnp.float32),
    )(parts.reshape(NWORK * 256, 16))
    return lreg[0, 0]
